# baseline (speedup 1.0000x reference)
import functools

import jax
import jax.numpy as jnp
from jax import lax
from jax.experimental import pallas as pl
from jax.experimental.pallas import tpu as pltpu
from jax.experimental.pallas import tpu_sc as plsc

NC = 2
NS = 16
L = 16
NW = NC * NS


def kernel(idx, tok_emb, pos_emb):
    B, T = idx.shape
    V, D = tok_emb.shape
    N = B * T
    TW = T // NW
    CH = 32
    NB = 3
    assert T % NW == 0 and D % L == 0 and TW % CH == 0 and B * TW >= NB * CH

    chunks = []
    for b in range(B):
        for h in range(TW // CH):
            chunks.append((b, b * TW + h * CH, h * CH))
    nch = len(chunks)

    mesh = plsc.VectorSubcoreMesh(
        core_axis_name="c", subcore_axis_name="s", num_cores=NC, num_subcores=NS
    )

    @functools.partial(
        pl.kernel,
        out_type=jax.ShapeDtypeStruct((B, T, D), jnp.float32),
        mesh=mesh,
        scratch_types=[
            pltpu.VMEM((B * TW,), jnp.int32),
            pltpu.VMEM((TW, D), jnp.float32),
            [pltpu.VMEM((CH, D), jnp.float32)] * NB,
            [pltpu.SemaphoreType.DMA] * NB,
            [pltpu.SemaphoreType.DMA] * NB,
            pltpu.SemaphoreType.DMA,
            pltpu.SemaphoreType.DMA,
        ],
    )
    def run(idx_hbm, tok_hbm, pos_hbm, out_hbm, idx_v, pos_v, rows,
            gsem, ssem, isem, psem):
        wid = lax.axis_index("s") * NC + lax.axis_index("c")
        t0 = wid * TW

        idone = [
            pltpu.async_copy(idx_hbm.at[b, pl.ds(t0, TW)],
                             idx_v.at[pl.ds(b * TW, TW)], isem)
            for b in range(B)
        ]
        pdone = pltpu.async_copy(pos_hbm.at[pl.ds(t0, TW)], pos_v, psem)
        for d in idone:
            d.wait()

        def gather(g):
            b = g % NB
            return pltpu.async_copy(
                tok_hbm.at[idx_v.at[pl.ds(chunks[g][1], CH)]],
                rows[b], gsem[b])

        gd = [None] * NB
        sd = [None] * NB
        LA = NB - 1
        for k in range(min(LA, nch)):
            gd[k % NB] = gather(k)
        pdone.wait()
        for g in range(nch):
            b = g % NB
            gd[b].wait()

            buf = rows[b]
            po = chunks[g][2]

            @plsc.parallel_loop(0, CH, 1, unroll=3)
            def _(i, buf=buf, po=po):
                for j in range(D // L):
                    sl = (i, pl.ds(j * L, L))
                    plsc.addupdate(buf.at[sl], pos_v[po + i, pl.ds(j * L, L)])

            sd[b] = pltpu.async_copy(
                buf, out_hbm.at[chunks[g][0], pl.ds(t0 + chunks[g][2], CH)],
                ssem[b])
            if g + LA < nch:
                nb = (g + LA) % NB
                if sd[nb] is not None:
                    sd[nb].wait()
                gd[nb] = gather(g + LA)
        for d in sd:
            if d is not None:
                d.wait()

    return run(idx, tok_emb, pos_emb)


# speedup vs baseline: 1.2886x; 1.2886x over previous
import functools

import jax
import jax.numpy as jnp
from jax import lax
from jax.experimental import pallas as pl
from jax.experimental.pallas import tpu as pltpu
from jax.experimental.pallas import tpu_sc as plsc

NC = 2
NS = 16
L = 16
NW = NC * NS


def kernel(idx, tok_emb, pos_emb):
    B, T = idx.shape
    V, D = tok_emb.shape
    TW = T // NW
    CH = 32
    NB = 4
    NPH = TW // CH
    assert T % NW == 0 and D % L == 0 and TW % CH == 0

    chunks = []
    for h in range(NPH):
        for b in range(B):
            chunks.append((b, b * TW + h * CH, h))
    nch = len(chunks)

    mesh = plsc.VectorSubcoreMesh(
        core_axis_name="c", subcore_axis_name="s", num_cores=NC, num_subcores=NS
    )

    @functools.partial(
        pl.kernel,
        out_type=jax.ShapeDtypeStruct((B, T, D), jnp.float32),
        mesh=mesh,
        scratch_types=[
            pltpu.VMEM((B * TW,), jnp.int32),
            pltpu.VMEM((CH, D), jnp.float32),
            [pltpu.VMEM((CH, D), jnp.float32)] * NB,
            [pltpu.SemaphoreType.DMA] * NB,
            [pltpu.SemaphoreType.DMA] * NB,
            pltpu.SemaphoreType.DMA,
            pltpu.SemaphoreType.DMA,
        ],
    )
    def run(idx_hbm, tok_hbm, pos_hbm, out_hbm, idx_v, pos_v, rows,
            gsem, ssem, isem, psem):
        wid = lax.axis_index("s") * NC + lax.axis_index("c")
        t0 = wid * TW

        idone = [
            pltpu.async_copy(idx_hbm.at[b, pl.ds(t0, TW)],
                             idx_v.at[pl.ds(b * TW, TW)], isem)
            for b in range(B)
        ]
        pd = pltpu.async_copy(pos_hbm.at[pl.ds(t0, CH)], pos_v, psem)
        for d in idone:
            d.wait()

        def gather(g):
            b = g % NB
            return pltpu.async_copy(
                tok_hbm.at[idx_v.at[pl.ds(chunks[g][1], CH)]],
                rows[b], gsem[b])

        gd = [None] * NB
        sd = [None] * NB
        LA = 2
        for k in range(min(LA, nch)):
            gd[k % NB] = gather(k)
        pd.wait()
        for g in range(nch):
            b = g % NB
            gd[b].wait()

            buf = rows[b]

            @plsc.parallel_loop(0, CH, 1, unroll=2)
            def _(i, buf=buf):
                for j in range(D // L):
                    sl = (i, pl.ds(j * L, L))
                    plsc.addupdate(buf.at[sl], pos_v[i, pl.ds(j * L, L)])

            refresh = (g % B == B - 1) and (chunks[g][2] + 1 < NPH)
            if refresh:
                pd = pltpu.async_copy(
                    pos_hbm.at[pl.ds(t0 + (chunks[g][2] + 1) * CH, CH)],
                    pos_v, psem)

            sd[b] = pltpu.async_copy(
                buf,
                out_hbm.at[chunks[g][0],
                           pl.ds(t0 + chunks[g][2] * CH, CH)],
                ssem[b])
            if g + LA < nch:
                nb = (g + LA) % NB
                if sd[nb] is not None:
                    sd[nb].wait()
                gd[nb] = gather(g + LA)
            if refresh:
                pd.wait()
        for d in sd:
            if d is not None:
                d.wait()

    return run(idx, tok_emb, pos_emb)


# speedup vs baseline: 1.2982x; 1.0074x over previous
import functools

import jax
import jax.numpy as jnp
from jax import lax
from jax.experimental import pallas as pl
from jax.experimental.pallas import tpu as pltpu
from jax.experimental.pallas import tpu_sc as plsc

NC = 2
NS = 16
L = 16
NW = NC * NS


def kernel(idx, tok_emb, pos_emb):
    B, T = idx.shape
    V, D = tok_emb.shape
    TW = T // NW
    CH = 32
    NB = 4
    NPH = TW // CH
    assert T % NW == 0 and D % L == 0 and TW % CH == 0

    chunks = []
    for h in range(NPH):
        for b in range(B):
            chunks.append((b, b * TW + h * CH, h))
    nch = len(chunks)

    mesh = plsc.VectorSubcoreMesh(
        core_axis_name="c", subcore_axis_name="s", num_cores=NC, num_subcores=NS
    )

    @functools.partial(
        pl.kernel,
        out_type=jax.ShapeDtypeStruct((B, T, D), jnp.float32),
        mesh=mesh,
        scratch_types=[
            pltpu.VMEM((B * TW,), jnp.int32),
            pltpu.VMEM((CH, D), jnp.float32),
            [pltpu.VMEM((CH, D), jnp.float32)] * NB,
            [pltpu.SemaphoreType.DMA] * NB,
            [pltpu.SemaphoreType.DMA] * NB,
            pltpu.SemaphoreType.DMA,
            pltpu.SemaphoreType.DMA,
        ],
    )
    def run(idx_hbm, tok_hbm, pos_hbm, out_hbm, idx_v, pos_v, rows,
            gsem, ssem, isem, psem):
        wid = lax.axis_index("s") * NC + lax.axis_index("c")
        t0 = wid * TW

        idone = [
            pltpu.async_copy(idx_hbm.at[b, pl.ds(t0, TW)],
                             idx_v.at[pl.ds(b * TW, TW)], isem)
            for b in range(B)
        ]
        pd = pltpu.async_copy(pos_hbm.at[pl.ds(t0, CH)], pos_v, psem)
        for d in idone:
            d.wait()

        def gather(g):
            b = g % NB
            return pltpu.async_copy(
                tok_hbm.at[idx_v.at[pl.ds(chunks[g][1], CH)]],
                rows[b], gsem[b])

        gd = [None] * NB
        sd = [None] * NB
        LA = 3
        for k in range(min(LA, nch)):
            gd[k % NB] = gather(k)
        pd.wait()
        for g in range(nch):
            b = g % NB
            gd[b].wait()

            buf = rows[b]

            @plsc.parallel_loop(0, CH, 1, unroll=2)
            def _(i, buf=buf):
                for j in range(D // L):
                    sl = (i, pl.ds(j * L, L))
                    plsc.addupdate(buf.at[sl], pos_v[i, pl.ds(j * L, L)])

            refresh = (g % B == B - 1) and (chunks[g][2] + 1 < NPH)
            if refresh:
                pd = pltpu.async_copy(
                    pos_hbm.at[pl.ds(t0 + (chunks[g][2] + 1) * CH, CH)],
                    pos_v, psem)

            sd[b] = pltpu.async_copy(
                buf,
                out_hbm.at[chunks[g][0],
                           pl.ds(t0 + chunks[g][2] * CH, CH)],
                ssem[b])
            if g + LA < nch:
                nb = (g + LA) % NB
                if sd[nb] is not None:
                    sd[nb].wait()
                gd[nb] = gather(g + LA)
            if refresh:
                pd.wait()
        for d in sd:
            if d is not None:
                d.wait()

    return run(idx, tok_emb, pos_emb)
